# Initial kernel scaffold; baseline (speedup 1.0000x reference)
import jax
import jax.numpy as jnp
from jax.experimental import pallas as pl


def kernel(x, edge_index, edge_weight, w_z, b_z, w_r, b_r, w_h, b_h, lin_w, lin_b):
    raise NotImplementedError("write your pallas kernel here")


# speedup vs baseline: 5.8655x; 5.8655x over previous
import functools

import jax
import jax.numpy as jnp
from jax import lax
from jax.experimental import pallas as pl
from jax.experimental.pallas import tpu as pltpu
from jax.experimental.pallas import tpu_sc as plsc

N = 10000
E = 320000
C = 128
OUT = 64
NPAD = 10240
BATCH = 128
TE = 20224
NBATCH = TE // BATCH
EH = TE * 16
ABATCH = 2528
NCHUNK = NPAD // 16

_MESH = plsc.VectorSubcoreMesh(core_axis_name="c", subcore_axis_name="s")
_SC_PARAMS = pltpu.CompilerParams(needs_layout_passes=False)


def _zero16():
    return jnp.zeros((16,), jnp.float32)


@functools.partial(
    pl.kernel,
    out_type=jax.ShapeDtypeStruct((2, NPAD), jnp.float32),
    mesh=_MESH,
    compiler_params=_SC_PARAMS,
    scratch_types=[
        pltpu.VMEM((BATCH,), jnp.int32),
        pltpu.VMEM((BATCH,), jnp.int32),
        pltpu.VMEM((BATCH,), jnp.float32),
        pltpu.VMEM((NCHUNK,), jnp.float32),
        pltpu.VMEM((2, NCHUNK), jnp.float32),
        pltpu.VMEM_SHARED((NPAD,), jnp.float32),
        pltpu.VMEM_SHARED((NPAD,), jnp.float32),
    ],
)
def _degrees(row_hbm, col_hbm, ew_hbm, inv_hbm,
             ridx, cidx, ew_b, zchunk, inv_b, deg_o_sh, deg_i_sh):
    c = lax.axis_index("c")
    s = lax.axis_index("s")

    @pl.when(c == 0)
    def _():
        def zbody(j, _):
            zchunk[pl.ds(j * 16, 16)] = _zero16()
            return 0
        lax.fori_loop(0, NCHUNK // 16, zbody, 0)
        pltpu.sync_copy(zchunk, deg_o_sh.at[pl.ds(s * NCHUNK, NCHUNK)])
        pltpu.sync_copy(zchunk, deg_i_sh.at[pl.ds(s * NCHUNK, NCHUNK)])
        plsc.subcore_barrier()

        base = s * TE
        def abody(b, _):
            off = base + b * BATCH
            pltpu.sync_copy(row_hbm.at[pl.ds(off, BATCH)], ridx)
            pltpu.sync_copy(col_hbm.at[pl.ds(off, BATCH)], cidx)
            pltpu.sync_copy(ew_hbm.at[pl.ds(off, BATCH)], ew_b)
            pltpu.sync_copy(ew_b, deg_o_sh.at[ridx], add=True)
            pltpu.sync_copy(ew_b, deg_i_sh.at[cidx], add=True)
            return 0
        lax.fori_loop(0, TE // BATCH, abody, 0)
        plsc.subcore_barrier()

        pltpu.sync_copy(deg_o_sh.at[pl.ds(s * NCHUNK, NCHUNK)], zchunk)
        def sbody(jj, d, buf):
            node = s * NCHUNK + jj * 16 + lax.iota(jnp.int32, 16)
            acc = buf[pl.ds(jj * 16, 16)]
            inv = jnp.where(node < N, 1.0 / acc, 0.0)
            inv_b[d, pl.ds(jj * 16, 16)] = inv
        lax.fori_loop(0, NCHUNK // 16, lambda jj, _: (sbody(jj, 0, zchunk), 0)[1], 0)
        pltpu.sync_copy(deg_i_sh.at[pl.ds(s * NCHUNK, NCHUNK)], zchunk)
        lax.fori_loop(0, NCHUNK // 16, lambda jj, _: (sbody(jj, 1, zchunk), 0)[1], 0)
        pltpu.sync_copy(inv_b, inv_hbm.at[:, pl.ds(s * NCHUNK, NCHUNK)])


@functools.partial(
    pl.kernel,
    out_type=jax.ShapeDtypeStruct((2, NPAD, C), jnp.float32),
    mesh=_MESH,
    compiler_params=_SC_PARAMS,
    scratch_types=[
        pltpu.VMEM((NPAD,), jnp.float32),
        pltpu.VMEM((BATCH,), jnp.int32),
        pltpu.VMEM((BATCH,), jnp.int32),
        pltpu.VMEM((BATCH,), jnp.int32),
        pltpu.VMEM((BATCH,), jnp.float32),
        pltpu.VMEM((BATCH, C), jnp.float32),
        pltpu.VMEM((BATCH, C), jnp.float32),
        pltpu.VMEM_SHARED((NPAD, C), jnp.float32),
        pltpu.SemaphoreType.DMA,
    ],
)
def _aggregate(x_hbm, src_hbm, dst_hbm, wrow_hbm, inv_hbm, acc_hbm,
               inv_l, sidx, didx, widx, wbuf, rows, zrow, acc_sh, gsem):
    c = lax.axis_index("c")
    s = lax.axis_index("s")

    pltpu.sync_copy(inv_hbm.at[c], inv_l)

    def zb(i, _):
        for r in range(8):
            zrow[i, pl.ds(r * 16, 16)] = _zero16()
        return 0
    lax.fori_loop(0, BATCH, zb, 0)
    for b in range(NCHUNK // BATCH):
        pltpu.sync_copy(zrow, acc_sh.at[pl.ds(s * NCHUNK + b * BATCH, BATCH)])
    plsc.subcore_barrier()

    half_base = c * EH + s * TE

    def body(b, _):
        off = half_base + b * BATCH
        pltpu.sync_copy(src_hbm.at[pl.ds(off, BATCH)], sidx)
        pltpu.sync_copy(dst_hbm.at[pl.ds(off, BATCH)], didx)
        pltpu.sync_copy(wrow_hbm.at[pl.ds(off, BATCH)], widx)
        pltpu.async_copy(x_hbm.at[sidx], rows, gsem).wait()

        def wb(k, _):
            iv = widx[pl.ds(k * 16, 16)]
            wbuf[pl.ds(k * 16, 16)] = plsc.load_gather(inv_l, [iv])
            return 0
        lax.fori_loop(0, BATCH // 16, wb, 0)

        def eb(e, _):
            ws = plsc.load_gather(wbuf, [jnp.full((16,), e, jnp.int32)])
            for r in range(8):
                rows[e, pl.ds(r * 16, 16)] = rows[e, pl.ds(r * 16, 16)] * ws
            return 0
        lax.fori_loop(0, BATCH, eb, 0)

        pltpu.sync_copy(rows, acc_sh.at[didx], add=True)
        return 0
    lax.fori_loop(0, NBATCH, body, 0)

    plsc.subcore_barrier()
    pltpu.sync_copy(acc_sh.at[pl.ds(s * NCHUNK, NCHUNK)],
                    acc_hbm.at[c, pl.ds(s * NCHUNK, NCHUNK)])


def _gates_body(x_ref, to_ref, ti_ref, w_ref, b_ref, lt_ref, lb_ref, o_ref):
    W = w_ref[...]
    pre = jnp.dot(x_ref[...], W[:C], preferred_element_type=jnp.float32)
    pre += jnp.dot(to_ref[0], W[C:2 * C], preferred_element_type=jnp.float32)
    pre += jnp.dot(ti_ref[0], W[2 * C:], preferred_element_type=jnp.float32)
    pre += b_ref[...]
    Z = jax.nn.sigmoid(pre[:, :C])
    Ht = jnp.tanh(pre[:, C:])
    h = jnp.maximum((1.0 - Z) * Ht, 0.0)
    o_ref[...] = jnp.dot(h, lt_ref[...], preferred_element_type=jnp.float32) \
        + lb_ref[...]


_BR = 512

_gates = pl.pallas_call(
    _gates_body,
    out_shape=jax.ShapeDtypeStruct((NPAD, OUT), jnp.float32),
    grid=(NPAD // _BR,),
    in_specs=[
        pl.BlockSpec((_BR, C), lambda i: (i, 0)),
        pl.BlockSpec((1, _BR, C), lambda i: (0, i, 0)),
        pl.BlockSpec((1, _BR, C), lambda i: (1, i, 0)),
        pl.BlockSpec((3 * C, 2 * C), lambda i: (0, 0)),
        pl.BlockSpec((1, 2 * C), lambda i: (0, 0)),
        pl.BlockSpec((C, OUT), lambda i: (0, 0)),
        pl.BlockSpec((1, OUT), lambda i: (0, 0)),
    ],
    out_specs=pl.BlockSpec((_BR, OUT), lambda i: (i, 0)),
)


def kernel(x, edge_index, edge_weight, w_z, b_z, w_r, b_r, w_h, b_h,
           lin_w, lin_b):
    row = edge_index[0].astype(jnp.int32)
    col = edge_index[1].astype(jnp.int32)
    order = jnp.lexsort((row, col))

    padE = EH - E
    padv = jnp.full((padE,), NPAD - 1, jnp.int32)
    src_all = jnp.concatenate([row, padv, col[order], padv])
    dst_all = jnp.concatenate([col, padv, row[order], padv])
    wrow_all = jnp.concatenate([row, padv, row, padv])
    row_pad = jnp.concatenate([row, padv])
    col_pad = jnp.concatenate([col, padv])
    ew_pad = jnp.concatenate([edge_weight, jnp.zeros((padE,), jnp.float32)])
    x_pad = jnp.concatenate([x, jnp.zeros((NPAD - N, C), x.dtype)])

    inv2 = _degrees(row_pad, col_pad, ew_pad)
    acc = _aggregate(x_pad, src_all, dst_all, wrow_all, inv2)

    wcat = jnp.concatenate([
        jnp.concatenate([w_z[0, 0][:C] + w_z[1, 0][:C],
                         w_h[0, 0][:C] + w_h[1, 0][:C]], axis=1),
        jnp.concatenate([w_z[0, 1][:C], w_h[0, 1][:C]], axis=1),
        jnp.concatenate([w_z[1, 1][:C], w_h[1, 1][:C]], axis=1),
    ], axis=0)
    bias = jnp.concatenate([b_z, b_h])[None]
    out = _gates(x_pad, acc, acc, wcat, bias, lin_w.T, lin_b[None])
    return out[:N]


# speedup vs baseline: 6.2178x; 1.0601x over previous
import functools

import jax
import jax.numpy as jnp
from jax import lax
from jax.experimental import pallas as pl
from jax.experimental.pallas import tpu as pltpu
from jax.experimental.pallas import tpu_sc as plsc

N = 10000
E = 320000
C = 128
OUT = 64
NPAD = 10240
BATCH = 128
TE = 20224
NBATCH = TE // BATCH
EH = TE * 16
ABATCH = 2528
NCHUNK = NPAD // 16

_MESH = plsc.VectorSubcoreMesh(core_axis_name="c", subcore_axis_name="s")
_SC_PARAMS = pltpu.CompilerParams(needs_layout_passes=False)


def _zero16():
    return jnp.zeros((16,), jnp.float32)


@functools.partial(
    pl.kernel,
    out_type=jax.ShapeDtypeStruct((2, NPAD), jnp.float32),
    mesh=_MESH,
    compiler_params=_SC_PARAMS,
    scratch_types=[
        pltpu.VMEM((BATCH,), jnp.int32),
        pltpu.VMEM((BATCH,), jnp.int32),
        pltpu.VMEM((BATCH,), jnp.float32),
        pltpu.VMEM((NCHUNK,), jnp.float32),
        pltpu.VMEM((2, NCHUNK), jnp.float32),
        pltpu.VMEM_SHARED((NPAD,), jnp.float32),
        pltpu.VMEM_SHARED((NPAD,), jnp.float32),
    ],
)
def _degrees(row_hbm, col_hbm, ew_hbm, inv_hbm,
             ridx, cidx, ew_b, zchunk, inv_b, deg_o_sh, deg_i_sh):
    c = lax.axis_index("c")
    s = lax.axis_index("s")

    @pl.when(c == 0)
    def _():
        def zbody(j, _):
            zchunk[pl.ds(j * 16, 16)] = _zero16()
            return 0
        lax.fori_loop(0, NCHUNK // 16, zbody, 0)
        pltpu.sync_copy(zchunk, deg_o_sh.at[pl.ds(s * NCHUNK, NCHUNK)])
        pltpu.sync_copy(zchunk, deg_i_sh.at[pl.ds(s * NCHUNK, NCHUNK)])
        plsc.subcore_barrier()

        base = s * TE
        def abody(b, _):
            off = base + b * BATCH
            pltpu.sync_copy(row_hbm.at[pl.ds(off, BATCH)], ridx)
            pltpu.sync_copy(col_hbm.at[pl.ds(off, BATCH)], cidx)
            pltpu.sync_copy(ew_hbm.at[pl.ds(off, BATCH)], ew_b)
            pltpu.sync_copy(ew_b, deg_o_sh.at[ridx], add=True)
            pltpu.sync_copy(ew_b, deg_i_sh.at[cidx], add=True)
            return 0
        lax.fori_loop(0, TE // BATCH, abody, 0)
        plsc.subcore_barrier()

        pltpu.sync_copy(deg_o_sh.at[pl.ds(s * NCHUNK, NCHUNK)], zchunk)
        def sbody(jj, d, buf):
            node = s * NCHUNK + jj * 16 + lax.iota(jnp.int32, 16)
            acc = buf[pl.ds(jj * 16, 16)]
            inv = jnp.where(node < N, 1.0 / acc, 0.0)
            inv_b[d, pl.ds(jj * 16, 16)] = inv
        lax.fori_loop(0, NCHUNK // 16, lambda jj, _: (sbody(jj, 0, zchunk), 0)[1], 0)
        pltpu.sync_copy(deg_i_sh.at[pl.ds(s * NCHUNK, NCHUNK)], zchunk)
        lax.fori_loop(0, NCHUNK // 16, lambda jj, _: (sbody(jj, 1, zchunk), 0)[1], 0)
        pltpu.sync_copy(inv_b, inv_hbm.at[:, pl.ds(s * NCHUNK, NCHUNK)])


@functools.partial(
    pl.kernel,
    out_type=jax.ShapeDtypeStruct((2, NPAD, C), jnp.float32),
    mesh=_MESH,
    compiler_params=_SC_PARAMS,
    scratch_types=[
        pltpu.VMEM((NPAD,), jnp.float32),
        pltpu.VMEM((BATCH,), jnp.int32),
        pltpu.VMEM((BATCH,), jnp.int32),
        pltpu.VMEM((BATCH,), jnp.int32),
        pltpu.VMEM((BATCH,), jnp.float32),
        pltpu.VMEM((BATCH, C), jnp.float32),
        pltpu.VMEM((BATCH, C), jnp.float32),
        pltpu.VMEM_SHARED((NPAD, C), jnp.float32),
        pltpu.SemaphoreType.DMA,
    ],
)
def _aggregate(x_hbm, src_hbm, dst_hbm, wrow_hbm, inv_hbm, acc_hbm,
               inv_l, sidx, didx, widx, wbuf, rows, zrow, acc_sh, gsem):
    c = lax.axis_index("c")
    s = lax.axis_index("s")

    pltpu.sync_copy(inv_hbm.at[c], inv_l)

    def zb(i, _):
        for r in range(8):
            zrow[i, pl.ds(r * 16, 16)] = _zero16()
        return 0
    lax.fori_loop(0, BATCH, zb, 0)
    for b in range(NCHUNK // BATCH):
        pltpu.sync_copy(zrow, acc_sh.at[pl.ds(s * NCHUNK + b * BATCH, BATCH)])
    plsc.subcore_barrier()

    half_base = c * EH + s * TE

    def body(b, _):
        off = half_base + b * BATCH
        pltpu.sync_copy(src_hbm.at[pl.ds(off, BATCH)], sidx)
        pltpu.sync_copy(dst_hbm.at[pl.ds(off, BATCH)], didx)
        pltpu.sync_copy(wrow_hbm.at[pl.ds(off, BATCH)], widx)
        pltpu.async_copy(x_hbm.at[sidx], rows, gsem).wait()

        def wb(k, _):
            iv = widx[pl.ds(k * 16, 16)]
            wbuf[pl.ds(k * 16, 16)] = plsc.load_gather(inv_l, [iv])
            return 0
        lax.fori_loop(0, BATCH // 16, wb, 0)

        def eb(e, _):
            ws = plsc.load_gather(wbuf, [jnp.full((16,), e, jnp.int32)])
            for r in range(8):
                rows[e, pl.ds(r * 16, 16)] = rows[e, pl.ds(r * 16, 16)] * ws
            return 0
        lax.fori_loop(0, BATCH, eb, 0)

        pltpu.sync_copy(rows, acc_sh.at[didx], add=True)
        return 0
    lax.fori_loop(0, NBATCH, body, 0)

    plsc.subcore_barrier()
    pltpu.sync_copy(acc_sh.at[pl.ds(s * NCHUNK, NCHUNK)],
                    acc_hbm.at[c, pl.ds(s * NCHUNK, NCHUNK)])


def _gates_body(x_ref, to_ref, ti_ref, w_ref, b_ref, lt_ref, lb_ref, o_ref):
    W = w_ref[...]
    pre = jnp.dot(x_ref[...], W[:C], preferred_element_type=jnp.float32)
    pre += jnp.dot(to_ref[0], W[C:2 * C], preferred_element_type=jnp.float32)
    pre += jnp.dot(ti_ref[0], W[2 * C:], preferred_element_type=jnp.float32)
    pre += b_ref[...]
    Z = jax.nn.sigmoid(pre[:, :C])
    Ht = jnp.tanh(pre[:, C:])
    h = jnp.maximum((1.0 - Z) * Ht, 0.0)
    o_ref[...] = jnp.dot(h, lt_ref[...], preferred_element_type=jnp.float32) \
        + lb_ref[...]


_BR = 512

_gates = pl.pallas_call(
    _gates_body,
    out_shape=jax.ShapeDtypeStruct((NPAD, OUT), jnp.float32),
    grid=(NPAD // _BR,),
    in_specs=[
        pl.BlockSpec((_BR, C), lambda i: (i, 0)),
        pl.BlockSpec((1, _BR, C), lambda i: (0, i, 0)),
        pl.BlockSpec((1, _BR, C), lambda i: (1, i, 0)),
        pl.BlockSpec((3 * C, 2 * C), lambda i: (0, 0)),
        pl.BlockSpec((1, 2 * C), lambda i: (0, 0)),
        pl.BlockSpec((C, OUT), lambda i: (0, 0)),
        pl.BlockSpec((1, OUT), lambda i: (0, 0)),
    ],
    out_specs=pl.BlockSpec((_BR, OUT), lambda i: (i, 0)),
)


def kernel(x, edge_index, edge_weight, w_z, b_z, w_r, b_r, w_h, b_h,
           lin_w, lin_b):
    row = edge_index[0].astype(jnp.int32)
    col = edge_index[1].astype(jnp.int32)
    order = jnp.argsort(col * 16384 + row)

    padE = EH - E
    padv = jnp.full((padE,), NPAD - 1, jnp.int32)
    src_all = jnp.concatenate([row, padv, col[order], padv])
    dst_all = jnp.concatenate([col, padv, row[order], padv])
    wrow_all = jnp.concatenate([row, padv, row, padv])
    row_pad = jnp.concatenate([row, padv])
    col_pad = jnp.concatenate([col, padv])
    ew_pad = jnp.concatenate([edge_weight, jnp.zeros((padE,), jnp.float32)])
    x_pad = jnp.concatenate([x, jnp.zeros((NPAD - N, C), x.dtype)])

    inv2 = _degrees(row_pad, col_pad, ew_pad)
    acc = _aggregate(x_pad, src_all, dst_all, wrow_all, inv2)

    wcat = jnp.concatenate([
        jnp.concatenate([w_z[0, 0][:C] + w_z[1, 0][:C],
                         w_h[0, 0][:C] + w_h[1, 0][:C]], axis=1),
        jnp.concatenate([w_z[0, 1][:C], w_h[0, 1][:C]], axis=1),
        jnp.concatenate([w_z[1, 1][:C], w_h[1, 1][:C]], axis=1),
    ], axis=0)
    bias = jnp.concatenate([b_z, b_h])[None]
    out = _gates(x_pad, acc, acc, wcat, bias, lin_w.T, lin_b[None])
    return out[:N]
